# Initial kernel scaffold; baseline (speedup 1.0000x reference)
import jax
import jax.numpy as jnp
from jax.experimental import pallas as pl


def kernel(node_feature_A, node_feature_B, edge_index_ab, edge_index_ba, batch_A, batch_B, W1l_ab, W1r_ab, b1_ab, W2l_ab, W2r_ab, b2_ab, W1l_ba, W1r_ba, b1_ba, W2l_ba, W2r_ba, b2_ba, g1_A, be1_A, g2_A, be2_A, g1_B, be1_B, g2_B, be2_B, Wg1, bg1, Wg2, bg2, Wg3, bg3):
    raise NotImplementedError("write your pallas kernel here")


# speedup vs baseline: 3.4250x; 3.4250x over previous
import functools

import jax
import jax.numpy as jnp
from jax import lax
from jax.experimental import pallas as pl
from jax.experimental.pallas import tpu as pltpu
from jax.experimental.pallas import tpu_sc as plsc

N = 10000
D = 128
H = 64
E = 320000
G = 64
EPS = 1e-5

NC = 2
NS = 16
NW = NC * NS
CH = 128
K = 8
TILE_E = 10240
EPAD = NW * TILE_E
ROWS_PER_TILE = TILE_E // CH
OUTER = ROWS_PER_TILE // K
NPAD = 10112
RPT = NPAD // NS


def _make_seg_kernel(with_counts):
    mesh = plsc.VectorSubcoreMesh(
        core_axis_name="c", subcore_axis_name="s", num_cores=NC, num_subcores=NS
    )
    outs = [
        jax.ShapeDtypeStruct((NC, NPAD, H), jnp.float32),
        jax.ShapeDtypeStruct((NC, NPAD, H), jnp.float32),
    ]
    scratch = [
        pltpu.VMEM((K, CH), jnp.int32),
        pltpu.VMEM((K, CH), jnp.int32),
        pltpu.VMEM((CH, H), jnp.float32),
        pltpu.VMEM_SHARED((NPAD, H), jnp.float32),
        pltpu.VMEM_SHARED((NPAD, H), jnp.float32),
        pltpu.SemaphoreType.DMA,
    ]
    if with_counts:
        outs += [
            jax.ShapeDtypeStruct((NC, NPAD), jnp.float32),
            jax.ShapeDtypeStruct((NC, NPAD), jnp.float32),
        ]
        scratch += [
            pltpu.VMEM((CH,), jnp.float32),
            pltpu.VMEM_SHARED((NPAD,), jnp.float32),
            pltpu.VMEM_SHARED((NPAD,), jnp.float32),
        ]

    def body(*refs):
        (tA, tB, sAB, dAB, sBA, dBA, zero_nh, zero_n) = refs[:8]
        (pB, pA) = refs[8:10]
        if with_counts:
            (cB, cA) = refs[10:12]
            (idx_s, idx_d, rows, accB, accA, sem, ones_v, cntB, cntA) = refs[12:]
        else:
            (idx_s, idx_d, rows, accB, accA, sem) = refs[10:]

        cid = lax.axis_index("c")
        sid = lax.axis_index("s")
        wid = sid * NC + cid

        @pl.when(sid == 0)
        def _zero():
            pltpu.sync_copy(zero_nh, accB)
            pltpu.sync_copy(zero_nh, accA)
            if with_counts:
                pltpu.sync_copy(zero_n, cntB)
                pltpu.sync_copy(zero_n, cntA)

        if with_counts:
            for i in range(CH // 16):
                ones_v[pl.ds(i * 16, 16)] = jnp.ones((16,), jnp.float32)
        plsc.subcore_barrier()

        def run_dir(table, src2d, dst2d, acc, cnt):
            def outer(t, carry):
                r0 = pl.multiple_of(wid * ROWS_PER_TILE + t * K, K)
                pltpu.sync_copy(src2d.at[pl.ds(r0, K)], idx_s)
                pltpu.sync_copy(dst2d.at[pl.ds(r0, K)], idx_d)
                for j in range(K):
                    pltpu.async_copy(table.at[idx_s.at[j]], rows, sem).wait()
                    pltpu.sync_copy(rows, acc.at[idx_d.at[j]], add=True)
                    if with_counts:
                        pltpu.sync_copy(ones_v, cnt.at[idx_d.at[j]], add=True)
                return carry
            lax.fori_loop(0, OUTER, outer, 0)

        run_dir(tA, sAB, dAB, accB, cntB if with_counts else None)
        run_dir(tB, sBA, dBA, accA, cntA if with_counts else None)
        plsc.subcore_barrier()

        sl = pl.ds(sid * RPT, RPT)
        pltpu.sync_copy(accB.at[sl], pB.at[cid, sl])
        pltpu.sync_copy(accA.at[sl], pA.at[cid, sl])
        if with_counts:
            @pl.when(sid == 0)
            def _wcnt():
                pltpu.sync_copy(cntB, cB.at[cid])
                pltpu.sync_copy(cntA, cA.at[cid])

    return pl.kernel(body, out_type=outs, mesh=mesh, scratch_types=scratch,
                     compiler_params=pltpu.CompilerParams(
                         use_tc_tiling_on_sc=False),
                     name="seg_sum_cnt" if with_counts else "seg_sum")


_seg_with_counts = _make_seg_kernel(True)
_seg_no_counts = _make_seg_kernel(False)


def _pad_edges(ei):
    src = jnp.concatenate(
        [ei[0].astype(jnp.int32), jnp.zeros((EPAD - E,), jnp.int32)])
    dst = jnp.concatenate(
        [ei[1].astype(jnp.int32), jnp.full((EPAD - E,), N, jnp.int32)])
    return src.reshape(EPAD // CH, CH), dst.reshape(EPAD // CH, CH)


def _lrelu(x):
    return jnp.where(x >= 0, x, 0.01 * x)


def _bn(x, g, b):
    m = jnp.mean(x, axis=0)
    v = jnp.var(x, axis=0)
    return (x - m) / jnp.sqrt(v + EPS) * g + b


def kernel(node_feature_A, node_feature_B, edge_index_ab, edge_index_ba,
           batch_A, batch_B, W1l_ab, W1r_ab, b1_ab, W2l_ab, W2r_ab, b2_ab,
           W1l_ba, W1r_ba, b1_ba, W2l_ba, W2r_ba, b2_ba, g1_A, be1_A, g2_A,
           be2_A, g1_B, be1_B, g2_B, be2_B, Wg1, bg1, Wg2, bg2, Wg3, bg3):
    fA, fB = node_feature_A, node_feature_B
    sAB, dAB = _pad_edges(edge_index_ab)
    sBA, dBA = _pad_edges(edge_index_ba)
    zero_nh = jnp.zeros((NPAD, H), jnp.float32)
    zero_n = jnp.zeros((NPAD,), jnp.float32)

    tA1 = fA @ W1l_ab
    tB1 = fB @ W1l_ba
    pB, pA, cB, cA = _seg_with_counts(tA1, tB1, sAB, dAB, sBA, dBA,
                                      zero_nh, zero_n)
    cntB = jnp.maximum(cB[0] + cB[1], 1.0)[:N, None]
    cntA = jnp.maximum(cA[0] + cA[1], 1.0)[:N, None]
    hB = (pB[0] + pB[1])[:N] / cntB + b1_ab + fB @ W1r_ab
    hA = (pA[0] + pA[1])[:N] / cntA + b1_ba + fA @ W1r_ba
    hA = _lrelu(_bn(hA, g1_A, be1_A))
    hB = _lrelu(_bn(hB, g1_B, be1_B))

    tA2 = hA @ W2l_ab
    tB2 = hB @ W2l_ba
    pB2, pA2 = _seg_no_counts(tA2, tB2, sAB, dAB, sBA, dBA, zero_nh, zero_n)
    h2B = (pB2[0] + pB2[1])[:N] / cntB + b2_ab + hB @ W2r_ab
    h2A = (pA2[0] + pA2[1])[:N] / cntA + b2_ba + hA @ W2r_ba
    h2A = _lrelu(_bn(h2A, g2_A, be2_A))
    h2B = _lrelu(_bn(h2B, g2_B, be2_B))

    onehotA = (batch_A[None, :] == jnp.arange(G, dtype=batch_A.dtype)[:, None])
    onehotB = (batch_B[None, :] == jnp.arange(G, dtype=batch_B.dtype)[:, None])
    onehotA = onehotA.astype(jnp.float32)
    onehotB = onehotB.astype(jnp.float32)
    gA = (onehotA @ h2A) / jnp.maximum(onehotA.sum(axis=1, keepdims=True), 1.0)
    gB = (onehotB @ h2B) / jnp.maximum(onehotB.sum(axis=1, keepdims=True), 1.0)
    emb = jnp.concatenate([gA, gB], axis=1)
    p = emb @ Wg1 + bg1
    p = p @ Wg2 + bg2
    p = p @ Wg3 + bg3
    return jax.nn.log_softmax(p, axis=1)


# speedup vs baseline: 3.4849x; 1.0175x over previous
import functools

import jax
import jax.numpy as jnp
from jax import lax
from jax.experimental import pallas as pl
from jax.experimental.pallas import tpu as pltpu
from jax.experimental.pallas import tpu_sc as plsc

N = 10000
D = 128
H = 64
E = 320000
G = 64
EPS = 1e-5

NC = 2
NS = 16
NW = NC * NS
CH = 128
K = 8
TILE_E = 10240
EPAD = NW * TILE_E
ROWS_PER_TILE = TILE_E // CH
OUTER = ROWS_PER_TILE // K
NPAD = 10112
RPT = NPAD // NS


def _make_seg_kernel(with_counts):
    mesh = plsc.VectorSubcoreMesh(
        core_axis_name="c", subcore_axis_name="s", num_cores=NC, num_subcores=NS
    )
    outs = [
        jax.ShapeDtypeStruct((NC, NPAD, H), jnp.float32),
        jax.ShapeDtypeStruct((NC, NPAD, H), jnp.float32),
    ]
    scratch = [
        pltpu.VMEM((K, CH), jnp.int32),
        pltpu.VMEM((K, CH), jnp.int32),
        pltpu.VMEM((CH, H), jnp.float32),
        pltpu.VMEM_SHARED((NPAD, H), jnp.float32),
        pltpu.VMEM_SHARED((NPAD, H), jnp.float32),
        pltpu.SemaphoreType.DMA,
    ]
    if with_counts:
        outs += [
            jax.ShapeDtypeStruct((NC, NPAD), jnp.float32),
            jax.ShapeDtypeStruct((NC, NPAD), jnp.float32),
        ]
        scratch += [
            pltpu.VMEM((CH,), jnp.float32),
            pltpu.VMEM_SHARED((NPAD,), jnp.float32),
            pltpu.VMEM_SHARED((NPAD,), jnp.float32),
        ]

    def body(*refs):
        (tA, tB, sAB, dAB, sBA, dBA, zero_nh, zero_n) = refs[:8]
        (pB, pA) = refs[8:10]
        if with_counts:
            (cB, cA) = refs[10:12]
            (idx_s, idx_d, rows, accB, accA, sem, ones_v, cntB, cntA) = refs[12:]
        else:
            (idx_s, idx_d, rows, accB, accA, sem) = refs[10:]

        cid = lax.axis_index("c")
        sid = lax.axis_index("s")
        wid = sid * NC + cid

        @pl.when(sid == 0)
        def _zero():
            pltpu.sync_copy(zero_nh, accB)
            pltpu.sync_copy(zero_nh, accA)
            if with_counts:
                pltpu.sync_copy(zero_n, cntB)
                pltpu.sync_copy(zero_n, cntA)

        if with_counts:
            for i in range(CH // 16):
                ones_v[pl.ds(i * 16, 16)] = jnp.ones((16,), jnp.float32)
        plsc.subcore_barrier()

        def run_dir(table, src2d, dst2d, acc, cnt):
            def outer(t, carry):
                r0 = pl.multiple_of(wid * ROWS_PER_TILE + t * K, K)
                pltpu.sync_copy(src2d.at[pl.ds(r0, K)], idx_s)
                pltpu.sync_copy(dst2d.at[pl.ds(r0, K)], idx_d)
                for j in range(K):
                    pltpu.async_copy(table.at[idx_s.at[j]], rows, sem).wait()
                    pltpu.sync_copy(rows, acc.at[idx_d.at[j]], add=True)
                    if with_counts:
                        pltpu.sync_copy(ones_v, cnt.at[idx_d.at[j]], add=True)
                return carry
            lax.fori_loop(0, OUTER, outer, 0)

        run_dir(tA, sAB, dAB, accB, cntB if with_counts else None)
        run_dir(tB, sBA, dBA, accA, cntA if with_counts else None)
        plsc.subcore_barrier()

        sl = pl.ds(sid * RPT, RPT)
        pltpu.sync_copy(accB.at[sl], pB.at[cid, sl])
        pltpu.sync_copy(accA.at[sl], pA.at[cid, sl])
        if with_counts:
            @pl.when(sid == 0)
            def _wcnt():
                pltpu.sync_copy(cntB, cB.at[cid])
                pltpu.sync_copy(cntA, cA.at[cid])

    return pl.kernel(body, out_type=outs, mesh=mesh, scratch_types=scratch,
                     compiler_params=pltpu.CompilerParams(
                         use_tc_tiling_on_sc=False),
                     name="seg_sum_cnt" if with_counts else "seg_sum")


_seg_with_counts = _make_seg_kernel(True)
_seg_no_counts = _make_seg_kernel(False)


def _pad_edges(ei):
    src = jnp.concatenate(
        [ei[0].astype(jnp.int32), jnp.zeros((EPAD - E,), jnp.int32)])
    dst = jnp.concatenate(
        [ei[1].astype(jnp.int32), jnp.full((EPAD - E,), N, jnp.int32)])
    return src.reshape(EPAD // CH, CH), dst.reshape(EPAD // CH, CH)


def _plrelu(x):
    return jnp.where(x >= 0, x, 0.01 * x)


def _bn_lrelu(x, g, b):
    m = jnp.mean(x, axis=0, keepdims=True)
    v = jnp.mean((x - m) * (x - m), axis=0, keepdims=True)
    return _plrelu((x - m) * lax.rsqrt(v + EPS) * g + b)


_TC_PARAMS = pltpu.CompilerParams(vmem_limit_bytes=110 * 2**20)

def _stage_pre(fA, fB, W1l_ab, W1r_ab, b1_ab, W1l_ba, W1r_ba, b1_ba):
    def body(fA_r, fB_r, wla_r, wra_r, ba_r, wlb_r, wrb_r, bb_r,
             tA1_r, tB1_r, rB1_r, rA1_r):
        fa = fA_r[...]
        fb = fB_r[...]
        tA1_r[...] = jnp.dot(fa, wla_r[...], preferred_element_type=jnp.float32)
        tB1_r[...] = jnp.dot(fb, wlb_r[...], preferred_element_type=jnp.float32)
        rB1_r[...] = jnp.dot(fb, wra_r[...], preferred_element_type=jnp.float32) + ba_r[...]
        rA1_r[...] = jnp.dot(fa, wrb_r[...], preferred_element_type=jnp.float32) + bb_r[...]

    out = [jax.ShapeDtypeStruct((N, H), jnp.float32)] * 4
    return pl.pallas_call(body, out_shape=out, compiler_params=_TC_PARAMS)(
        fA, fB, W1l_ab, W1r_ab, b1_ab.reshape(1, H), W1l_ba, W1r_ba,
        b1_ba.reshape(1, H))


def _stage_mid_one(pX, icX, rX1, g1, be1, Wl_next, Wr_next, b_next):
    def body(pX_r, icX_r, rX1_r, g1_r, be1_r, wl_r, wr_r, bn_r, tX2_r, rX2_r):
        hX = (pX_r[0, :N, :] + pX_r[1, :N, :]) * icX_r[...] + rX1_r[...]
        hX = _bn_lrelu(hX, g1_r[...], be1_r[...])
        tX2_r[...] = jnp.dot(hX, wl_r[...], preferred_element_type=jnp.float32)
        rX2_r[...] = jnp.dot(hX, wr_r[...], preferred_element_type=jnp.float32) + bn_r[...]

    out = [jax.ShapeDtypeStruct((N, H), jnp.float32)] * 2
    return pl.pallas_call(body, out_shape=out)(
        pX, icX, rX1, g1.reshape(1, H), be1.reshape(1, H),
        Wl_next, Wr_next, b_next.reshape(1, H))


def _stage_post_one(pX2, icX, rX2, g2, be2, batch_X):
    def body(pX_r, icX_r, rX2_r, g2_r, be2_r, bX_r, gX_r):
        h2X = (pX_r[0, :N, :] + pX_r[1, :N, :]) * icX_r[...] + rX2_r[...]
        h2X = _bn_lrelu(h2X, g2_r[...], be2_r[...])
        gids = lax.broadcasted_iota(jnp.int32, (G, N), 0)
        oh = (bX_r[...] == gids).astype(jnp.float32)
        gX_r[...] = jnp.dot(oh, h2X, preferred_element_type=jnp.float32) / jnp.maximum(
            jnp.sum(oh, axis=1, keepdims=True), 1.0)

    out = jax.ShapeDtypeStruct((G, H), jnp.float32)
    return pl.pallas_call(body, out_shape=out)(
        pX2, icX, rX2, g2.reshape(1, H), be2.reshape(1, H),
        batch_X.reshape(1, N).astype(jnp.int32))


def _stage_head(gA, gB, Wg1, bg1, Wg2, bg2, Wg3, bg3):
    def body(gA_r, gB_r, wg1_r, b1_r, wg2_r, b2_r, wg3_r, b3_r, out_r):
        emb = jnp.concatenate([gA_r[...], gB_r[...]], axis=1)
        p = jnp.dot(emb, wg1_r[...], preferred_element_type=jnp.float32) + b1_r[...]
        p = jnp.dot(p, wg2_r[...], preferred_element_type=jnp.float32) + b2_r[...]
        p = jnp.dot(p, wg3_r[...], preferred_element_type=jnp.float32) + b3_r[...]
        pm = jnp.max(p, axis=1, keepdims=True)
        lse = jnp.log(jnp.sum(jnp.exp(p - pm), axis=1, keepdims=True)) + pm
        out_r[...] = p - lse

    out = jax.ShapeDtypeStruct((G, 8), jnp.float32)
    return pl.pallas_call(body, out_shape=out)(
        gA, gB, Wg1, bg1.reshape(1, H), Wg2, bg2.reshape(1, 16),
        Wg3, bg3.reshape(1, 8))


def kernel(node_feature_A, node_feature_B, edge_index_ab, edge_index_ba,
           batch_A, batch_B, W1l_ab, W1r_ab, b1_ab, W2l_ab, W2r_ab, b2_ab,
           W1l_ba, W1r_ba, b1_ba, W2l_ba, W2r_ba, b2_ba, g1_A, be1_A, g2_A,
           be2_A, g1_B, be1_B, g2_B, be2_B, Wg1, bg1, Wg2, bg2, Wg3, bg3):
    fA, fB = node_feature_A, node_feature_B
    sAB, dAB = _pad_edges(edge_index_ab)
    sBA, dBA = _pad_edges(edge_index_ba)
    zero_nh = jnp.zeros((NPAD, H), jnp.float32)
    zero_n = jnp.zeros((NPAD,), jnp.float32)

    tA1, tB1, rB1, rA1 = _stage_pre(fA, fB, W1l_ab, W1r_ab, b1_ab,
                                    W1l_ba, W1r_ba, b1_ba)
    pB, pA, cB, cA = _seg_with_counts(tA1, tB1, sAB, dAB, sBA, dBA,
                                      zero_nh, zero_n)
    icB = jnp.broadcast_to(
        (1.0 / jnp.maximum(cB[0] + cB[1], 1.0))[:N, None], (N, H))
    icA = jnp.broadcast_to(
        (1.0 / jnp.maximum(cA[0] + cA[1], 1.0))[:N, None], (N, H))
    tB2, rB2 = _stage_mid_one(pB, icB, rB1, g1_B, be1_B, W2l_ba, W2r_ab, b2_ab)
    tA2, rA2 = _stage_mid_one(pA, icA, rA1, g1_A, be1_A, W2l_ab, W2r_ba, b2_ba)

    pB2, pA2 = _seg_no_counts(tA2, tB2, sAB, dAB, sBA, dBA, zero_nh, zero_n)
    gB = _stage_post_one(pB2, icB, rB2, g2_B, be2_B, batch_B)
    gA = _stage_post_one(pA2, icA, rA2, g2_A, be2_A, batch_A)
    return _stage_head(gA, gB, Wg1, bg1, Wg2, bg2, Wg3, bg3)


# speedup vs baseline: 5.9396x; 1.7044x over previous
import functools

import jax
import jax.numpy as jnp
from jax import lax
from jax.experimental import pallas as pl
from jax.experimental.pallas import tpu as pltpu
from jax.experimental.pallas import tpu_sc as plsc

N = 10000
D = 128
H = 64
E = 320000
G = 64
EPS = 1e-5

NC = 2
NS = 16
NW = NC * NS
CH = 128
K = 8
NB = 8
TILE_E = 20480
EPAD = NS * TILE_E
ROWS_PER_TILE = TILE_E // CH
OUTER = ROWS_PER_TILE // K
NPAD = 10112
RPT = NPAD // NS


def _make_seg_kernel(with_counts):
    mesh = plsc.VectorSubcoreMesh(
        core_axis_name="c", subcore_axis_name="s", num_cores=NC, num_subcores=NS
    )
    outs = [
        jax.ShapeDtypeStruct((NPAD, H), jnp.float32),
        jax.ShapeDtypeStruct((NPAD, H), jnp.float32),
    ]
    scratch = [
        pltpu.VMEM((2, K, CH), jnp.int32),
        pltpu.VMEM((2, K, CH), jnp.int32),
        pltpu.VMEM((NB, CH, H), jnp.float32),
        pltpu.VMEM_SHARED((NPAD, H), jnp.float32),
        pltpu.SemaphoreType.DMA,
        pltpu.SemaphoreType.DMA,
        pltpu.SemaphoreType.DMA,
        pltpu.SemaphoreType.DMA,
    ]
    if with_counts:
        outs += [
            jax.ShapeDtypeStruct((NPAD,), jnp.float32),
            jax.ShapeDtypeStruct((NPAD,), jnp.float32),
        ]
        scratch += [
            pltpu.VMEM((CH,), jnp.float32),
            pltpu.VMEM_SHARED((NPAD,), jnp.float32),
            pltpu.SemaphoreType.DMA,
        ]

    HB = NB // 2

    def body(*refs):
        (tA, tB, sAB, dAB, sBA, dBA, zero_nh, zero_n) = refs[:8]
        (pB, pA) = refs[8:10]
        if with_counts:
            (cB, cA) = refs[10:12]
            (isv, idv, rows, acc, gs0, gs1, ss0, ss1,
             ones_v, cnt, csem) = refs[12:]
        else:
            (isv, idv, rows, acc, gs0, gs1, ss0, ss1) = refs[10:]
            cnt = None

        cid = lax.axis_index("c")
        sid = lax.axis_index("s")

        @pl.when(sid == 0)
        def _zero():
            pltpu.sync_copy(zero_nh, acc)
            if with_counts:
                pltpu.sync_copy(zero_n, cnt)

        if with_counts:
            for i in range(CH // 16):
                ones_v[pl.ds(i * 16, 16)] = jnp.ones((16,), jnp.float32)
        plsc.subcore_barrier()

        def run_dir(table, src2d, dst2d):
            base = sid * ROWS_PER_TILE

            def ldidx(i, p):
                r0 = base + i * K
                pltpu.sync_copy(src2d.at[pl.ds(r0, K)], isv.at[p])
                pltpu.sync_copy(dst2d.at[pl.ds(r0, K)], idv.at[p])

            def gissue(p, j, b, sem):
                pltpu.async_copy(table.at[isv.at[p, j]], rows.at[b], sem)

            def gwait(b, sem):
                pltpu.make_async_copy(
                    table.at[isv.at[0, 0]], rows.at[b], sem).wait()

            def sissue(p, j, b, sem):
                pltpu.async_copy(rows.at[b], acc.at[idv.at[p, j]], sem,
                                 add=True)

            def swait(b, sem):
                pltpu.make_async_copy(
                    rows.at[b], acc.at[idv.at[0, 0]], sem).wait()

            def cissue(p, j):
                pltpu.async_copy(ones_v, cnt.at[idv.at[p, j]], csem, add=True)

            def cwait():
                pltpu.make_async_copy(ones_v, cnt.at[idv.at[0, 0]], csem).wait()

            def group(i, p, issue_next, load_next):
                for j in range(HB):
                    gwait(j, gs0)
                for j in range(HB):
                    sissue(p, j, j, ss0)
                    if with_counts:
                        cissue(p, j)
                for j in range(HB):
                    gwait(HB + j, gs1)
                for j in range(HB):
                    sissue(p, HB + j, HB + j, ss1)
                    if with_counts:
                        cissue(p, HB + j)
                for j in range(HB):
                    swait(j, ss0)
                if issue_next:
                    for j in range(HB):
                        gissue(1 - p, j, j, gs0)
                for j in range(HB):
                    swait(HB + j, ss1)
                if issue_next:
                    for j in range(HB):
                        gissue(1 - p, HB + j, HB + j, gs1)
                if with_counts:
                    for j in range(K):
                        cwait()
                if load_next:
                    ldidx(i + 2, p)

            ldidx(0, 0)
            ldidx(1, 1)
            for j in range(HB):
                gissue(0, j, j, gs0)
            for j in range(HB):
                gissue(0, HB + j, HB + j, gs1)

            def outer(fi, carry):
                group(2 * fi, 0, True, True)
                group(2 * fi + 1, 1, True, True)
                return carry
            lax.fori_loop(0, OUTER // 2 - 1, outer, 0)
            group(OUTER - 2, 0, True, False)
            group(OUTER - 1, 1, False, False)

        @pl.when(cid == 0)
        def _dir_ab():
            run_dir(tA, sAB, dAB)

        @pl.when(cid == 1)
        def _dir_ba():
            run_dir(tB, sBA, dBA)

        plsc.subcore_barrier()

        sl = pl.ds(sid * RPT, RPT)

        @pl.when(cid == 0)
        def _wb_b():
            pltpu.sync_copy(acc.at[sl], pB.at[sl])
            if with_counts:
                @pl.when(sid == 0)
                def _wc_b():
                    pltpu.sync_copy(cnt, cB)

        @pl.when(cid == 1)
        def _wb_a():
            pltpu.sync_copy(acc.at[sl], pA.at[sl])
            if with_counts:
                @pl.when(sid == 0)
                def _wc_a():
                    pltpu.sync_copy(cnt, cA)

    return pl.kernel(body, out_type=outs, mesh=mesh, scratch_types=scratch,
                     compiler_params=pltpu.CompilerParams(
                         use_tc_tiling_on_sc=False),
                     name="seg_sum_cnt" if with_counts else "seg_sum")


_seg_with_counts = _make_seg_kernel(True)
_seg_no_counts = _make_seg_kernel(False)


def _pad_edges(ei):
    src = jnp.concatenate(
        [ei[0].astype(jnp.int32), jnp.zeros((EPAD - E,), jnp.int32)])
    dst = jnp.concatenate(
        [ei[1].astype(jnp.int32), jnp.full((EPAD - E,), N, jnp.int32)])
    return src.reshape(EPAD // CH, CH), dst.reshape(EPAD // CH, CH)


def _plrelu(x):
    return jnp.where(x >= 0, x, 0.01 * x)


def _bn_lrelu(x, g, b):
    m = jnp.mean(x, axis=0, keepdims=True)
    v = jnp.mean((x - m) * (x - m), axis=0, keepdims=True)
    return _plrelu((x - m) * lax.rsqrt(v + EPS) * g + b)


_TC_PARAMS = pltpu.CompilerParams(vmem_limit_bytes=110 * 2**20)

def _stage_pre(fA, fB, W1l_ab, W1r_ab, b1_ab, W1l_ba, W1r_ba, b1_ba):
    def body(fA_r, fB_r, wla_r, wra_r, ba_r, wlb_r, wrb_r, bb_r,
             tA1_r, tB1_r, rB1_r, rA1_r):
        fa = fA_r[...]
        fb = fB_r[...]
        tA1_r[...] = jnp.dot(fa, wla_r[...], preferred_element_type=jnp.float32)
        tB1_r[...] = jnp.dot(fb, wlb_r[...], preferred_element_type=jnp.float32)
        rB1_r[...] = jnp.dot(fb, wra_r[...], preferred_element_type=jnp.float32) + ba_r[...]
        rA1_r[...] = jnp.dot(fa, wrb_r[...], preferred_element_type=jnp.float32) + bb_r[...]

    out = [jax.ShapeDtypeStruct((N, H), jnp.float32)] * 4
    return pl.pallas_call(body, out_shape=out, compiler_params=_TC_PARAMS)(
        fA, fB, W1l_ab, W1r_ab, b1_ab.reshape(1, H), W1l_ba, W1r_ba,
        b1_ba.reshape(1, H))


def _stage_mid_one(pX, icX, rX1, g1, be1, Wl_next, Wr_next, b_next):
    def body(pX_r, icX_r, rX1_r, g1_r, be1_r, wl_r, wr_r, bn_r, tX2_r, rX2_r):
        hX = pX_r[:N, :] * icX_r[...] + rX1_r[...]
        hX = _bn_lrelu(hX, g1_r[...], be1_r[...])
        tX2_r[...] = jnp.dot(hX, wl_r[...], preferred_element_type=jnp.float32)
        rX2_r[...] = jnp.dot(hX, wr_r[...], preferred_element_type=jnp.float32) + bn_r[...]

    out = [jax.ShapeDtypeStruct((N, H), jnp.float32)] * 2
    return pl.pallas_call(body, out_shape=out)(
        pX, icX, rX1, g1.reshape(1, H), be1.reshape(1, H),
        Wl_next, Wr_next, b_next.reshape(1, H))


def _stage_post_one(pX2, icX, rX2, g2, be2, batch_X):
    def body(pX_r, icX_r, rX2_r, g2_r, be2_r, bX_r, gX_r):
        h2X = pX_r[:N, :] * icX_r[...] + rX2_r[...]
        h2X = _bn_lrelu(h2X, g2_r[...], be2_r[...])
        gids = lax.broadcasted_iota(jnp.int32, (G, N), 0)
        oh = (bX_r[...] == gids).astype(jnp.float32)
        gX_r[...] = jnp.dot(oh, h2X, preferred_element_type=jnp.float32) / jnp.maximum(
            jnp.sum(oh, axis=1, keepdims=True), 1.0)

    out = jax.ShapeDtypeStruct((G, H), jnp.float32)
    return pl.pallas_call(body, out_shape=out)(
        pX2, icX, rX2, g2.reshape(1, H), be2.reshape(1, H),
        batch_X.reshape(1, N).astype(jnp.int32))


def _stage_head(gA, gB, Wg1, bg1, Wg2, bg2, Wg3, bg3):
    def body(gA_r, gB_r, wg1_r, b1_r, wg2_r, b2_r, wg3_r, b3_r, out_r):
        emb = jnp.concatenate([gA_r[...], gB_r[...]], axis=1)
        p = jnp.dot(emb, wg1_r[...], preferred_element_type=jnp.float32) + b1_r[...]
        p = jnp.dot(p, wg2_r[...], preferred_element_type=jnp.float32) + b2_r[...]
        p = jnp.dot(p, wg3_r[...], preferred_element_type=jnp.float32) + b3_r[...]
        pm = jnp.max(p, axis=1, keepdims=True)
        lse = jnp.log(jnp.sum(jnp.exp(p - pm), axis=1, keepdims=True)) + pm
        out_r[...] = p - lse

    out = jax.ShapeDtypeStruct((G, 8), jnp.float32)
    return pl.pallas_call(body, out_shape=out)(
        gA, gB, Wg1, bg1.reshape(1, H), Wg2, bg2.reshape(1, 16),
        Wg3, bg3.reshape(1, 8))


def kernel(node_feature_A, node_feature_B, edge_index_ab, edge_index_ba,
           batch_A, batch_B, W1l_ab, W1r_ab, b1_ab, W2l_ab, W2r_ab, b2_ab,
           W1l_ba, W1r_ba, b1_ba, W2l_ba, W2r_ba, b2_ba, g1_A, be1_A, g2_A,
           be2_A, g1_B, be1_B, g2_B, be2_B, Wg1, bg1, Wg2, bg2, Wg3, bg3):
    fA, fB = node_feature_A, node_feature_B
    sAB, dAB = _pad_edges(edge_index_ab)
    sBA, dBA = _pad_edges(edge_index_ba)
    zero_nh = jnp.zeros((NPAD, H), jnp.float32)
    zero_n = jnp.zeros((NPAD,), jnp.float32)

    tA1, tB1, rB1, rA1 = _stage_pre(fA, fB, W1l_ab, W1r_ab, b1_ab,
                                    W1l_ba, W1r_ba, b1_ba)
    pB, pA, cB, cA = _seg_with_counts(tA1, tB1, sAB, dAB, sBA, dBA,
                                      zero_nh, zero_n)
    icB = jnp.broadcast_to(
        (1.0 / jnp.maximum(cB, 1.0))[:N, None], (N, H))
    icA = jnp.broadcast_to(
        (1.0 / jnp.maximum(cA, 1.0))[:N, None], (N, H))
    tB2, rB2 = _stage_mid_one(pB, icB, rB1, g1_B, be1_B, W2l_ba, W2r_ab, b2_ab)
    tA2, rA2 = _stage_mid_one(pA, icA, rA1, g1_A, be1_A, W2l_ab, W2r_ba, b2_ba)

    pB2, pA2 = _seg_no_counts(tA2, tB2, sAB, dAB, sBA, dBA, zero_nh, zero_n)
    gB = _stage_post_one(pB2, icB, rB2, g2_B, be2_B, batch_B)
    gA = _stage_post_one(pA2, icA, rA2, g2_A, be2_A, batch_A)
    return _stage_head(gA, gB, Wg1, bg1, Wg2, bg2, Wg3, bg3)


# speedup vs baseline: 6.1212x; 1.0306x over previous
import functools

import jax
import jax.numpy as jnp
from jax import lax
from jax.experimental import pallas as pl
from jax.experimental.pallas import tpu as pltpu
from jax.experimental.pallas import tpu_sc as plsc

N = 10000
D = 128
H = 64
E = 320000
G = 64
EPS = 1e-5

NC = 2
NS = 16
NW = NC * NS
CH = 128
Q = 2
NB = 4 * Q
TILE_E = 20480
EPAD = NS * TILE_E
ROWS_PER_TILE = TILE_E // CH
OCTETS = ROWS_PER_TILE // Q // 8
NPAD = 10112
RPT = NPAD // NS


def _make_seg_kernel(with_counts):
    mesh = plsc.VectorSubcoreMesh(
        core_axis_name="c", subcore_axis_name="s", num_cores=NC, num_subcores=NS
    )
    outs = [
        jax.ShapeDtypeStruct((NPAD, H), jnp.float32),
        jax.ShapeDtypeStruct((NPAD, H), jnp.float32),
    ]
    scratch = [
        pltpu.VMEM((2, 4 * Q, CH), jnp.int32),
        pltpu.VMEM((2, 4 * Q, CH), jnp.int32),
        pltpu.VMEM((NB, CH, H), jnp.float32),
        pltpu.VMEM_SHARED((NPAD, H), jnp.float32),
        pltpu.SemaphoreType.DMA,
        pltpu.SemaphoreType.DMA,
        pltpu.SemaphoreType.DMA,
        pltpu.SemaphoreType.DMA,
        pltpu.SemaphoreType.DMA,
        pltpu.SemaphoreType.DMA,
        pltpu.SemaphoreType.DMA,
        pltpu.SemaphoreType.DMA,
    ]
    if with_counts:
        outs += [
            jax.ShapeDtypeStruct((NPAD,), jnp.float32),
            jax.ShapeDtypeStruct((NPAD,), jnp.float32),
        ]
        scratch += [
            pltpu.VMEM((CH,), jnp.float32),
            pltpu.VMEM_SHARED((NPAD,), jnp.float32),
            pltpu.SemaphoreType.DMA,
        ]

    def body(*refs):
        (tA, tB, sAB, dAB, sBA, dBA, zero_nh, zero_n) = refs[:8]
        (pB, pA) = refs[8:10]
        if with_counts:
            (cB, cA) = refs[10:12]
            (isv, idv, rows, acc, gs0, gs1, gs2, gs3, ss0, ss1, ss2, ss3,
             ones_v, cnt, csem) = refs[12:]
        else:
            (isv, idv, rows, acc, gs0, gs1, gs2, gs3,
             ss0, ss1, ss2, ss3) = refs[10:]
            cnt = None

        cid = lax.axis_index("c")
        sid = lax.axis_index("s")

        @pl.when(sid == 0)
        def _zero():
            pltpu.sync_copy(zero_nh, acc)
            if with_counts:
                pltpu.sync_copy(zero_n, cnt)

        if with_counts:
            for i in range(CH // 16):
                ones_v[pl.ds(i * 16, 16)] = jnp.ones((16,), jnp.float32)
        plsc.subcore_barrier()

        def run_dir(table, src2d, dst2d):
            base = sid * ROWS_PER_TILE
            gs = [gs0, gs1, gs2, gs3]
            ss = [ss0, ss1, ss2, ss3]

            def ldbatch(b, p):
                r0 = base + b * 4 * Q
                pltpu.sync_copy(src2d.at[pl.ds(r0, 4 * Q)], isv.at[p])
                pltpu.sync_copy(dst2d.at[pl.ds(r0, 4 * Q)], idv.at[p])

            def gissue(p, row, qj):
                for q in range(Q):
                    pltpu.async_copy(table.at[isv.at[p, row + q]],
                                     rows.at[qj * Q + q], gs[qj])

            def gwait(qj):
                for q in range(Q):
                    pltpu.make_async_copy(table.at[isv.at[0, 0]],
                                          rows.at[qj * Q + q], gs[qj]).wait()

            def sissue(p, row, qj):
                for q in range(Q):
                    pltpu.async_copy(rows.at[qj * Q + q],
                                     acc.at[idv.at[p, row + q]], ss[qj],
                                     add=True)
                    if with_counts:
                        pltpu.async_copy(ones_v, cnt.at[idv.at[p, row + q]],
                                         csem, add=True)

            def swait(qj):
                for q in range(Q):
                    pltpu.make_async_copy(rows.at[qj * Q + q],
                                          acc.at[idv.at[0, 0]], ss[qj]).wait()

            def cwait(n):
                for _ in range(n):
                    pltpu.make_async_copy(ones_v, cnt.at[idv.at[0, 0]],
                                          csem).wait()

            def step(o, u, do_swait, do_gnext, do_ld):
                pb = (u // 4) % 2
                row = (u % 4) * Q
                qj = u % 4
                gwait(qj)
                sissue(pb, row, qj)
                if do_swait:
                    swait((u + 2) % 4)
                    if with_counts:
                        cwait(Q)
                if do_gnext:
                    gissue(((u + 2) // 4) % 2, ((u + 2) % 4) * Q, (u + 2) % 4)
                if do_ld and u % 4 == 1:
                    ldbatch(2 * o + (u + 3) // 4, ((u + 3) // 4) % 2)

            ldbatch(0, 0)
            ldbatch(1, 1)
            gissue(0, 0, 0)
            gissue(0, Q, 1)

            zero_o = jnp.int32(0)
            for u in range(8):
                step(zero_o, u, u >= 2, True, u == 5)

            def outer(o, carry):
                for u in range(8):
                    step(o, u, True, True, True)
                return carry
            lax.fori_loop(1, OCTETS - 1, outer, 0)

            last_o = jnp.int32(OCTETS - 1)
            for u in range(8):
                step(last_o, u, True, u < 6, u == 1)

            swait(2)
            swait(3)
            if with_counts:
                cwait(2 * Q)

        @pl.when(cid == 0)
        def _dir_ab():
            run_dir(tA, sAB, dAB)

        @pl.when(cid == 1)
        def _dir_ba():
            run_dir(tB, sBA, dBA)

        plsc.subcore_barrier()

        sl = pl.ds(sid * RPT, RPT)

        @pl.when(cid == 0)
        def _wb_b():
            pltpu.sync_copy(acc.at[sl], pB.at[sl])
            if with_counts:
                @pl.when(sid == 0)
                def _wc_b():
                    pltpu.sync_copy(cnt, cB)

        @pl.when(cid == 1)
        def _wb_a():
            pltpu.sync_copy(acc.at[sl], pA.at[sl])
            if with_counts:
                @pl.when(sid == 0)
                def _wc_a():
                    pltpu.sync_copy(cnt, cA)

    return pl.kernel(body, out_type=outs, mesh=mesh, scratch_types=scratch,
                     compiler_params=pltpu.CompilerParams(
                         use_tc_tiling_on_sc=False),
                     name="seg_sum_cnt" if with_counts else "seg_sum")


_seg_with_counts = _make_seg_kernel(True)
_seg_no_counts = _make_seg_kernel(False)


def _pad_edges(ei):
    src = jnp.concatenate(
        [ei[0].astype(jnp.int32), jnp.zeros((EPAD - E,), jnp.int32)])
    dst = jnp.concatenate(
        [ei[1].astype(jnp.int32), jnp.full((EPAD - E,), N, jnp.int32)])
    return src.reshape(EPAD // CH, CH), dst.reshape(EPAD // CH, CH)


def _plrelu(x):
    return jnp.where(x >= 0, x, 0.01 * x)


def _bn_lrelu(x, g, b):
    m = jnp.mean(x, axis=0, keepdims=True)
    v = jnp.mean((x - m) * (x - m), axis=0, keepdims=True)
    return _plrelu((x - m) * lax.rsqrt(v + EPS) * g + b)


_TC_PARAMS = pltpu.CompilerParams(vmem_limit_bytes=110 * 2**20)

def _stage_pre(fA, fB, W1l_ab, W1r_ab, b1_ab, W1l_ba, W1r_ba, b1_ba):
    def body(fA_r, fB_r, wla_r, wra_r, ba_r, wlb_r, wrb_r, bb_r,
             tA1_r, tB1_r, rB1_r, rA1_r):
        fa = fA_r[...]
        fb = fB_r[...]
        tA1_r[...] = jnp.dot(fa, wla_r[...], preferred_element_type=jnp.float32)
        tB1_r[...] = jnp.dot(fb, wlb_r[...], preferred_element_type=jnp.float32)
        rB1_r[...] = jnp.dot(fb, wra_r[...], preferred_element_type=jnp.float32) + ba_r[...]
        rA1_r[...] = jnp.dot(fa, wrb_r[...], preferred_element_type=jnp.float32) + bb_r[...]

    out = [jax.ShapeDtypeStruct((N, H), jnp.float32)] * 4
    return pl.pallas_call(body, out_shape=out, compiler_params=_TC_PARAMS)(
        fA, fB, W1l_ab, W1r_ab, b1_ab.reshape(1, H), W1l_ba, W1r_ba,
        b1_ba.reshape(1, H))


def _stage_mid_one(pX, icX, rX1, g1, be1, Wl_next, Wr_next, b_next):
    def body(pX_r, icX_r, rX1_r, g1_r, be1_r, wl_r, wr_r, bn_r, tX2_r, rX2_r):
        hX = pX_r[:N, :] * icX_r[...] + rX1_r[...]
        hX = _bn_lrelu(hX, g1_r[...], be1_r[...])
        tX2_r[...] = jnp.dot(hX, wl_r[...], preferred_element_type=jnp.float32)
        rX2_r[...] = jnp.dot(hX, wr_r[...], preferred_element_type=jnp.float32) + bn_r[...]

    out = [jax.ShapeDtypeStruct((N, H), jnp.float32)] * 2
    return pl.pallas_call(body, out_shape=out)(
        pX, icX, rX1, g1.reshape(1, H), be1.reshape(1, H),
        Wl_next, Wr_next, b_next.reshape(1, H))


def _stage_post_one(pX2, icX, rX2, g2, be2, batch_X):
    def body(pX_r, icX_r, rX2_r, g2_r, be2_r, bX_r, gX_r):
        h2X = pX_r[:N, :] * icX_r[...] + rX2_r[...]
        h2X = _bn_lrelu(h2X, g2_r[...], be2_r[...])
        gids = lax.broadcasted_iota(jnp.int32, (G, N), 0)
        oh = (bX_r[...] == gids).astype(jnp.float32)
        gX_r[...] = jnp.dot(oh, h2X, preferred_element_type=jnp.float32) / jnp.maximum(
            jnp.sum(oh, axis=1, keepdims=True), 1.0)

    out = jax.ShapeDtypeStruct((G, H), jnp.float32)
    return pl.pallas_call(body, out_shape=out)(
        pX2, icX, rX2, g2.reshape(1, H), be2.reshape(1, H),
        batch_X.reshape(1, N).astype(jnp.int32))


def _stage_head(gA, gB, Wg1, bg1, Wg2, bg2, Wg3, bg3):
    def body(gA_r, gB_r, wg1_r, b1_r, wg2_r, b2_r, wg3_r, b3_r, out_r):
        emb = jnp.concatenate([gA_r[...], gB_r[...]], axis=1)
        p = jnp.dot(emb, wg1_r[...], preferred_element_type=jnp.float32) + b1_r[...]
        p = jnp.dot(p, wg2_r[...], preferred_element_type=jnp.float32) + b2_r[...]
        p = jnp.dot(p, wg3_r[...], preferred_element_type=jnp.float32) + b3_r[...]
        pm = jnp.max(p, axis=1, keepdims=True)
        lse = jnp.log(jnp.sum(jnp.exp(p - pm), axis=1, keepdims=True)) + pm
        out_r[...] = p - lse

    out = jax.ShapeDtypeStruct((G, 8), jnp.float32)
    return pl.pallas_call(body, out_shape=out)(
        gA, gB, Wg1, bg1.reshape(1, H), Wg2, bg2.reshape(1, 16),
        Wg3, bg3.reshape(1, 8))


def kernel(node_feature_A, node_feature_B, edge_index_ab, edge_index_ba,
           batch_A, batch_B, W1l_ab, W1r_ab, b1_ab, W2l_ab, W2r_ab, b2_ab,
           W1l_ba, W1r_ba, b1_ba, W2l_ba, W2r_ba, b2_ba, g1_A, be1_A, g2_A,
           be2_A, g1_B, be1_B, g2_B, be2_B, Wg1, bg1, Wg2, bg2, Wg3, bg3):
    fA, fB = node_feature_A, node_feature_B
    sAB, dAB = _pad_edges(edge_index_ab)
    sBA, dBA = _pad_edges(edge_index_ba)
    zero_nh = jnp.zeros((NPAD, H), jnp.float32)
    zero_n = jnp.zeros((NPAD,), jnp.float32)

    tA1, tB1, rB1, rA1 = _stage_pre(fA, fB, W1l_ab, W1r_ab, b1_ab,
                                    W1l_ba, W1r_ba, b1_ba)
    pB, pA, cB, cA = _seg_with_counts(tA1, tB1, sAB, dAB, sBA, dBA,
                                      zero_nh, zero_n)
    icB = jnp.broadcast_to(
        (1.0 / jnp.maximum(cB, 1.0))[:N, None], (N, H))
    icA = jnp.broadcast_to(
        (1.0 / jnp.maximum(cA, 1.0))[:N, None], (N, H))
    tB2, rB2 = _stage_mid_one(pB, icB, rB1, g1_B, be1_B, W2l_ba, W2r_ab, b2_ab)
    tA2, rA2 = _stage_mid_one(pA, icA, rA1, g1_A, be1_A, W2l_ab, W2r_ba, b2_ba)

    pB2, pA2 = _seg_no_counts(tA2, tB2, sAB, dAB, sBA, dBA, zero_nh, zero_n)
    gB = _stage_post_one(pB2, icB, rB2, g2_B, be2_B, batch_B)
    gA = _stage_post_one(pA2, icA, rA2, g2_A, be2_A, batch_A)
    return _stage_head(gA, gB, Wg1, bg1, Wg2, bg2, Wg3, bg3)


# speedup vs baseline: 10.4162x; 1.7017x over previous
import functools

import jax
import jax.numpy as jnp
from jax import lax
from jax.experimental import pallas as pl
from jax.experimental.pallas import tpu as pltpu
from jax.experimental.pallas import tpu_sc as plsc

N = 10000
D = 128
H = 64
E = 320000
G = 64
EPS = 1e-5

NC = 2
NS = 16
NW = NC * NS
CH = 128
Q = 1
NB = 4 * Q
TILE_E = 20480
EPAD = NS * TILE_E
ROWS_PER_TILE = TILE_E // CH
OCTETS = ROWS_PER_TILE // Q // 8
NSTAGE = N // NS
NPAD = 10112
RPT = NPAD // NS


def _make_seg_kernel(with_counts):
    mesh = plsc.VectorSubcoreMesh(
        core_axis_name="c", subcore_axis_name="s", num_cores=NC, num_subcores=NS
    )
    outs = [
        jax.ShapeDtypeStruct((NPAD, H), jnp.float32),
        jax.ShapeDtypeStruct((NPAD, H), jnp.float32),
    ]
    scratch = [
        pltpu.VMEM((2, 4 * Q, CH), jnp.int32),
        pltpu.VMEM((2, 4 * Q, CH), jnp.int32),
        pltpu.VMEM((NB, CH, H), jnp.float32),
        pltpu.VMEM_SHARED((NPAD, H), jnp.float32),
        pltpu.VMEM_SHARED((N, H), jnp.float32),
        pltpu.SemaphoreType.DMA,
        pltpu.SemaphoreType.DMA,
        pltpu.SemaphoreType.DMA,
        pltpu.SemaphoreType.DMA,
        pltpu.SemaphoreType.DMA,
        pltpu.SemaphoreType.DMA,
        pltpu.SemaphoreType.DMA,
        pltpu.SemaphoreType.DMA,
    ]
    if with_counts:
        outs += [
            jax.ShapeDtypeStruct((NPAD,), jnp.float32),
            jax.ShapeDtypeStruct((NPAD,), jnp.float32),
        ]
        scratch += [
            pltpu.VMEM((CH,), jnp.float32),
            pltpu.VMEM_SHARED((NPAD,), jnp.float32),
            pltpu.SemaphoreType.DMA,
        ]

    def body(*refs):
        (tA, tB, sAB, dAB, sBA, dBA, zero_nh, zero_n) = refs[:8]
        (pB, pA) = refs[8:10]
        if with_counts:
            (cB, cA) = refs[10:12]
            (isv, idv, rows, acc, tbl, gs0, gs1, gs2, gs3, ss0, ss1, ss2,
             ss3, ones_v, cnt, csem) = refs[12:]
        else:
            (isv, idv, rows, acc, tbl, gs0, gs1, gs2, gs3,
             ss0, ss1, ss2, ss3) = refs[10:]
            cnt = None

        cid = lax.axis_index("c")
        sid = lax.axis_index("s")

        @pl.when(sid == 0)
        def _zero():
            pltpu.sync_copy(zero_nh, acc)
            if with_counts:
                pltpu.sync_copy(zero_n, cnt)

        tsl = pl.ds(sid * NSTAGE, NSTAGE)

        @pl.when(cid == 0)
        def _stage_a():
            pltpu.sync_copy(tA.at[tsl], tbl.at[tsl])

        @pl.when(cid == 1)
        def _stage_b():
            pltpu.sync_copy(tB.at[tsl], tbl.at[tsl])

        if with_counts:
            for i in range(CH // 16):
                ones_v[pl.ds(i * 16, 16)] = jnp.ones((16,), jnp.float32)
        plsc.subcore_barrier()

        def run_dir(src2d, dst2d):
            base = sid * ROWS_PER_TILE
            gs = [gs0, gs1, gs2, gs3]
            ss = [ss0, ss1, ss2, ss3]

            def ldbatch(b, p):
                r0 = base + b * 4 * Q
                pltpu.sync_copy(src2d.at[pl.ds(r0, 4 * Q)], isv.at[p])
                pltpu.sync_copy(dst2d.at[pl.ds(r0, 4 * Q)], idv.at[p])

            def gissue(p, row, qj):
                for q in range(Q):
                    pltpu.async_copy(tbl.at[isv.at[p, row + q]],
                                     rows.at[qj * Q + q], gs[qj])

            def gwait(qj):
                for q in range(Q):
                    pltpu.make_async_copy(tbl.at[isv.at[0, 0]],
                                          rows.at[qj * Q + q], gs[qj]).wait()

            def sissue(p, row, qj):
                for q in range(Q):
                    pltpu.async_copy(rows.at[qj * Q + q],
                                     acc.at[idv.at[p, row + q]], ss[qj],
                                     add=True)
                    if with_counts:
                        pltpu.async_copy(ones_v, cnt.at[idv.at[p, row + q]],
                                         csem, add=True)

            def swait(qj):
                for q in range(Q):
                    pltpu.make_async_copy(rows.at[qj * Q + q],
                                          acc.at[idv.at[0, 0]], ss[qj]).wait()

            def cwait(n):
                for _ in range(n):
                    pltpu.make_async_copy(ones_v, cnt.at[idv.at[0, 0]],
                                          csem).wait()

            def step(o, u, do_swait, do_gnext, do_ld):
                pb = (u // 4) % 2
                row = (u % 4) * Q
                qj = u % 4
                gwait(qj)
                sissue(pb, row, qj)
                if do_swait:
                    swait((u + 2) % 4)
                    if with_counts:
                        cwait(Q)
                if do_gnext:
                    gissue(((u + 2) // 4) % 2, ((u + 2) % 4) * Q, (u + 2) % 4)
                if do_ld and u % 4 == 1:
                    ldbatch(2 * o + (u + 3) // 4, ((u + 3) // 4) % 2)

            ldbatch(0, 0)
            ldbatch(1, 1)
            gissue(0, 0, 0)
            gissue(0, Q, 1)

            zero_o = jnp.int32(0)
            for u in range(8):
                step(zero_o, u, u >= 2, True, u == 5)

            def outer(o, carry):
                for u in range(8):
                    step(o, u, True, True, True)
                return carry
            lax.fori_loop(1, OCTETS - 1, outer, 0)

            last_o = jnp.int32(OCTETS - 1)
            for u in range(8):
                step(last_o, u, True, u < 6, u == 1)

            swait(2)
            swait(3)
            if with_counts:
                cwait(2 * Q)

        @pl.when(cid == 0)
        def _dir_ab():
            run_dir(sAB, dAB)

        @pl.when(cid == 1)
        def _dir_ba():
            run_dir(sBA, dBA)

        plsc.subcore_barrier()

        sl = pl.ds(sid * RPT, RPT)

        @pl.when(cid == 0)
        def _wb_b():
            pltpu.sync_copy(acc.at[sl], pB.at[sl])
            if with_counts:
                @pl.when(sid == 0)
                def _wc_b():
                    pltpu.sync_copy(cnt, cB)

        @pl.when(cid == 1)
        def _wb_a():
            pltpu.sync_copy(acc.at[sl], pA.at[sl])
            if with_counts:
                @pl.when(sid == 0)
                def _wc_a():
                    pltpu.sync_copy(cnt, cA)

    return pl.kernel(body, out_type=outs, mesh=mesh, scratch_types=scratch,
                     compiler_params=pltpu.CompilerParams(
                         use_tc_tiling_on_sc=False),
                     name="seg_sum_cnt" if with_counts else "seg_sum")


_seg_with_counts = _make_seg_kernel(True)
_seg_no_counts = _make_seg_kernel(False)


def _pad_edges(ei):
    src = jnp.concatenate(
        [ei[0].astype(jnp.int32), jnp.zeros((EPAD - E,), jnp.int32)])
    dst = jnp.concatenate(
        [ei[1].astype(jnp.int32), jnp.full((EPAD - E,), N, jnp.int32)])
    return src.reshape(EPAD // CH, CH), dst.reshape(EPAD // CH, CH)


def _plrelu(x):
    return jnp.where(x >= 0, x, 0.01 * x)


def _bn_lrelu(x, g, b):
    m = jnp.mean(x, axis=0, keepdims=True)
    v = jnp.mean((x - m) * (x - m), axis=0, keepdims=True)
    return _plrelu((x - m) * lax.rsqrt(v + EPS) * g + b)


_TC_PARAMS = pltpu.CompilerParams(vmem_limit_bytes=110 * 2**20)

def _stage_pre(fA, fB, W1l_ab, W1r_ab, b1_ab, W1l_ba, W1r_ba, b1_ba):
    def body(fA_r, fB_r, wla_r, wra_r, ba_r, wlb_r, wrb_r, bb_r,
             tA1_r, tB1_r, rB1_r, rA1_r):
        fa = fA_r[...]
        fb = fB_r[...]
        tA1_r[...] = jnp.dot(fa, wla_r[...], preferred_element_type=jnp.float32)
        tB1_r[...] = jnp.dot(fb, wlb_r[...], preferred_element_type=jnp.float32)
        rB1_r[...] = jnp.dot(fb, wra_r[...], preferred_element_type=jnp.float32) + ba_r[...]
        rA1_r[...] = jnp.dot(fa, wrb_r[...], preferred_element_type=jnp.float32) + bb_r[...]

    out = [jax.ShapeDtypeStruct((N, H), jnp.float32)] * 4
    return pl.pallas_call(body, out_shape=out, compiler_params=_TC_PARAMS)(
        fA, fB, W1l_ab, W1r_ab, b1_ab.reshape(1, H), W1l_ba, W1r_ba,
        b1_ba.reshape(1, H))


def _stage_mid_one(pX, icX, rX1, g1, be1, Wl_next, Wr_next, b_next):
    def body(pX_r, icX_r, rX1_r, g1_r, be1_r, wl_r, wr_r, bn_r, tX2_r, rX2_r):
        hX = pX_r[:N, :] * icX_r[...] + rX1_r[...]
        hX = _bn_lrelu(hX, g1_r[...], be1_r[...])
        tX2_r[...] = jnp.dot(hX, wl_r[...], preferred_element_type=jnp.float32)
        rX2_r[...] = jnp.dot(hX, wr_r[...], preferred_element_type=jnp.float32) + bn_r[...]

    out = [jax.ShapeDtypeStruct((N, H), jnp.float32)] * 2
    return pl.pallas_call(body, out_shape=out)(
        pX, icX, rX1, g1.reshape(1, H), be1.reshape(1, H),
        Wl_next, Wr_next, b_next.reshape(1, H))


def _stage_post_one(pX2, icX, rX2, g2, be2, batch_X):
    def body(pX_r, icX_r, rX2_r, g2_r, be2_r, bX_r, gX_r):
        h2X = pX_r[:N, :] * icX_r[...] + rX2_r[...]
        h2X = _bn_lrelu(h2X, g2_r[...], be2_r[...])
        gids = lax.broadcasted_iota(jnp.int32, (G, N), 0)
        oh = (bX_r[...] == gids).astype(jnp.float32)
        gX_r[...] = jnp.dot(oh, h2X, preferred_element_type=jnp.float32) / jnp.maximum(
            jnp.sum(oh, axis=1, keepdims=True), 1.0)

    out = jax.ShapeDtypeStruct((G, H), jnp.float32)
    return pl.pallas_call(body, out_shape=out)(
        pX2, icX, rX2, g2.reshape(1, H), be2.reshape(1, H),
        batch_X.reshape(1, N).astype(jnp.int32))


def _stage_head(gA, gB, Wg1, bg1, Wg2, bg2, Wg3, bg3):
    def body(gA_r, gB_r, wg1_r, b1_r, wg2_r, b2_r, wg3_r, b3_r, out_r):
        emb = jnp.concatenate([gA_r[...], gB_r[...]], axis=1)
        p = jnp.dot(emb, wg1_r[...], preferred_element_type=jnp.float32) + b1_r[...]
        p = jnp.dot(p, wg2_r[...], preferred_element_type=jnp.float32) + b2_r[...]
        p = jnp.dot(p, wg3_r[...], preferred_element_type=jnp.float32) + b3_r[...]
        pm = jnp.max(p, axis=1, keepdims=True)
        lse = jnp.log(jnp.sum(jnp.exp(p - pm), axis=1, keepdims=True)) + pm
        out_r[...] = p - lse

    out = jax.ShapeDtypeStruct((G, 8), jnp.float32)
    return pl.pallas_call(body, out_shape=out)(
        gA, gB, Wg1, bg1.reshape(1, H), Wg2, bg2.reshape(1, 16),
        Wg3, bg3.reshape(1, 8))


def kernel(node_feature_A, node_feature_B, edge_index_ab, edge_index_ba,
           batch_A, batch_B, W1l_ab, W1r_ab, b1_ab, W2l_ab, W2r_ab, b2_ab,
           W1l_ba, W1r_ba, b1_ba, W2l_ba, W2r_ba, b2_ba, g1_A, be1_A, g2_A,
           be2_A, g1_B, be1_B, g2_B, be2_B, Wg1, bg1, Wg2, bg2, Wg3, bg3):
    fA, fB = node_feature_A, node_feature_B
    sAB, dAB = _pad_edges(edge_index_ab)
    sBA, dBA = _pad_edges(edge_index_ba)
    zero_nh = jnp.zeros((NPAD, H), jnp.float32)
    zero_n = jnp.zeros((NPAD,), jnp.float32)

    tA1, tB1, rB1, rA1 = _stage_pre(fA, fB, W1l_ab, W1r_ab, b1_ab,
                                    W1l_ba, W1r_ba, b1_ba)
    pB, pA, cB, cA = _seg_with_counts(tA1, tB1, sAB, dAB, sBA, dBA,
                                      zero_nh, zero_n)
    icB = jnp.broadcast_to(
        (1.0 / jnp.maximum(cB, 1.0))[:N, None], (N, H))
    icA = jnp.broadcast_to(
        (1.0 / jnp.maximum(cA, 1.0))[:N, None], (N, H))
    tB2, rB2 = _stage_mid_one(pB, icB, rB1, g1_B, be1_B, W2l_ba, W2r_ab, b2_ab)
    tA2, rA2 = _stage_mid_one(pA, icA, rA1, g1_A, be1_A, W2l_ab, W2r_ba, b2_ba)

    pB2, pA2 = _seg_no_counts(tA2, tB2, sAB, dAB, sBA, dBA, zero_nh, zero_n)
    gB = _stage_post_one(pB2, icB, rB2, g2_B, be2_B, batch_B)
    gA = _stage_post_one(pA2, icA, rA2, g2_A, be2_A, batch_A)
    return _stage_head(gA, gB, Wg1, bg1, Wg2, bg2, Wg3, bg3)


# speedup vs baseline: 10.6408x; 1.0216x over previous
import functools

import jax
import jax.numpy as jnp
from jax import lax
from jax.experimental import pallas as pl
from jax.experimental.pallas import tpu as pltpu
from jax.experimental.pallas import tpu_sc as plsc

N = 10000
D = 128
H = 64
E = 320000
G = 64
EPS = 1e-5

NC = 2
NS = 16
NW = NC * NS
CH = 128
Q = 1
NB = 4 * Q
TILE_E = 20480
EPAD = NS * TILE_E
ROWS_PER_TILE = TILE_E // CH
OCTETS = ROWS_PER_TILE // Q // 8
NSTAGE = N // NS
NPAD = 10112
RPT = NPAD // NS


def _make_seg_kernel(with_counts):
    mesh = plsc.VectorSubcoreMesh(
        core_axis_name="c", subcore_axis_name="s", num_cores=NC, num_subcores=NS
    )
    outs = [
        jax.ShapeDtypeStruct((NPAD, H), jnp.float32),
        jax.ShapeDtypeStruct((NPAD, H), jnp.float32),
    ]
    scratch = [
        pltpu.VMEM((2, 4 * Q, CH), jnp.int32),
        pltpu.VMEM((2, 4 * Q, CH), jnp.int32),
        pltpu.VMEM((NB, CH, H), jnp.float32),
        pltpu.VMEM_SHARED((NPAD, H), jnp.float32),
        pltpu.VMEM_SHARED((N, H), jnp.float32),
        pltpu.SemaphoreType.DMA,
        pltpu.SemaphoreType.DMA,
        pltpu.SemaphoreType.DMA,
        pltpu.SemaphoreType.DMA,
        pltpu.SemaphoreType.DMA,
        pltpu.SemaphoreType.DMA,
        pltpu.SemaphoreType.DMA,
        pltpu.SemaphoreType.DMA,
    ]
    if with_counts:
        outs += [
            jax.ShapeDtypeStruct((NPAD,), jnp.float32),
            jax.ShapeDtypeStruct((NPAD,), jnp.float32),
        ]
        scratch += [
            pltpu.VMEM((CH,), jnp.float32),
            pltpu.VMEM_SHARED((NPAD,), jnp.float32),
            pltpu.SemaphoreType.DMA,
        ]

    def body(*refs):
        (tA, tB, sAB, dAB, sBA, dBA, zero_nh, zero_n) = refs[:8]
        (pB, pA) = refs[8:10]
        if with_counts:
            (cB, cA) = refs[10:12]
            (isv, idv, rows, acc, tbl, gs0, gs1, gs2, gs3, ss0, ss1, ss2,
             ss3, ones_v, cnt, csem) = refs[12:]
        else:
            (isv, idv, rows, acc, tbl, gs0, gs1, gs2, gs3,
             ss0, ss1, ss2, ss3) = refs[10:]
            cnt = None

        cid = lax.axis_index("c")
        sid = lax.axis_index("s")

        @pl.when(sid == 0)
        def _zero():
            pltpu.sync_copy(zero_nh, acc)
            if with_counts:
                pltpu.sync_copy(zero_n, cnt)

        tsl = pl.ds(sid * NSTAGE, NSTAGE)

        @pl.when(cid == 0)
        def _stage_a():
            pltpu.sync_copy(tA.at[tsl], tbl.at[tsl])

        @pl.when(cid == 1)
        def _stage_b():
            pltpu.sync_copy(tB.at[tsl], tbl.at[tsl])

        if with_counts:
            for i in range(CH // 16):
                ones_v[pl.ds(i * 16, 16)] = jnp.ones((16,), jnp.float32)
        plsc.subcore_barrier()

        def run_dir(src2d, dst2d):
            base = sid * ROWS_PER_TILE
            gs = [gs0, gs1, gs2, gs3]
            ss = [ss0, ss1, ss2, ss3]

            def ldbatch(b, p):
                r0 = base + b * 4 * Q
                pltpu.sync_copy(src2d.at[pl.ds(r0, 4 * Q)], isv.at[p])
                pltpu.sync_copy(dst2d.at[pl.ds(r0, 4 * Q)], idv.at[p])

            def gissue(p, row, qj):
                for q in range(Q):
                    pltpu.async_copy(tbl.at[isv.at[p, row + q]],
                                     rows.at[qj * Q + q], gs[qj])

            def gwait(qj):
                for q in range(Q):
                    pltpu.make_async_copy(tbl.at[isv.at[0, 0]],
                                          rows.at[qj * Q + q], gs[qj]).wait()

            def sissue(p, row, qj):
                for q in range(Q):
                    pltpu.async_copy(rows.at[qj * Q + q],
                                     acc.at[idv.at[p, row + q]], ss[qj],
                                     add=True)
                    if with_counts:
                        pltpu.async_copy(ones_v, cnt.at[idv.at[p, row + q]],
                                         csem, add=True)

            def swait(qj):
                for q in range(Q):
                    pltpu.make_async_copy(rows.at[qj * Q + q],
                                          acc.at[idv.at[0, 0]], ss[qj]).wait()

            def cwait(n):
                for _ in range(n):
                    pltpu.make_async_copy(ones_v, cnt.at[idv.at[0, 0]],
                                          csem).wait()

            def step(o, u, do_swait, do_gnext, do_ld):
                pb = (u // 4) % 2
                row = (u % 4) * Q
                qj = u % 4
                gwait(qj)
                sissue(pb, row, qj)
                if do_swait:
                    swait((u + 2) % 4)
                    if with_counts:
                        cwait(Q)
                if do_gnext:
                    gissue(((u + 2) // 4) % 2, ((u + 2) % 4) * Q, (u + 2) % 4)
                if do_ld and u % 4 == 1:
                    ldbatch(2 * o + (u + 3) // 4, ((u + 3) // 4) % 2)

            ldbatch(0, 0)
            ldbatch(1, 1)
            gissue(0, 0, 0)
            gissue(0, Q, 1)

            zero_o = jnp.int32(0)
            for u in range(8):
                step(zero_o, u, u >= 2, True, u == 5)

            def outer(o, carry):
                for u in range(8):
                    step(o, u, True, True, True)
                return carry
            lax.fori_loop(1, OCTETS - 1, outer, 0)

            last_o = jnp.int32(OCTETS - 1)
            for u in range(8):
                step(last_o, u, True, u < 6, u == 1)

            swait(2)
            swait(3)
            if with_counts:
                cwait(2 * Q)

        @pl.when(cid == 0)
        def _dir_ab():
            run_dir(sAB, dAB)

        @pl.when(cid == 1)
        def _dir_ba():
            run_dir(sBA, dBA)

        plsc.subcore_barrier()

        sl = pl.ds(sid * RPT, RPT)

        @pl.when(cid == 0)
        def _wb_b():
            pltpu.sync_copy(acc.at[sl], pB.at[sl])
            if with_counts:
                @pl.when(sid == 0)
                def _wc_b():
                    pltpu.sync_copy(cnt, cB)

        @pl.when(cid == 1)
        def _wb_a():
            pltpu.sync_copy(acc.at[sl], pA.at[sl])
            if with_counts:
                @pl.when(sid == 0)
                def _wc_a():
                    pltpu.sync_copy(cnt, cA)

    return pl.kernel(body, out_type=outs, mesh=mesh, scratch_types=scratch,
                     compiler_params=pltpu.CompilerParams(
                         use_tc_tiling_on_sc=False),
                     name="seg_sum_cnt" if with_counts else "seg_sum")


_seg_with_counts = _make_seg_kernel(True)
_seg_no_counts = _make_seg_kernel(False)


def _pad_edges(ei):
    src = jnp.concatenate(
        [ei[0].astype(jnp.int32), jnp.zeros((EPAD - E,), jnp.int32)])
    dst = jnp.concatenate(
        [ei[1].astype(jnp.int32), jnp.full((EPAD - E,), N, jnp.int32)])
    return src.reshape(EPAD // CH, CH), dst.reshape(EPAD // CH, CH)


def _plrelu(x):
    return jnp.where(x >= 0, x, 0.01 * x)


def _bn_lrelu(x, g, b):
    m = jnp.mean(x, axis=0, keepdims=True)
    v = jnp.mean((x - m) * (x - m), axis=0, keepdims=True)
    return _plrelu((x - m) * lax.rsqrt(v + EPS) * g + b)


_TC_PARAMS = pltpu.CompilerParams(vmem_limit_bytes=110 * 2**20)

def _tables1(fA, fB, W1l_ab, W1l_ba):
    def body(fA_r, fB_r, wla_r, wlb_r, tA1_r, tB1_r):
        tA1_r[...] = jnp.dot(fA_r[...], wla_r[...],
                             preferred_element_type=jnp.float32)
        tB1_r[...] = jnp.dot(fB_r[...], wlb_r[...],
                             preferred_element_type=jnp.float32)

    out = [jax.ShapeDtypeStruct((N, H), jnp.float32)] * 2
    return pl.pallas_call(body, out_shape=out)(fA, fB, W1l_ab, W1l_ba)


def _rterms1(fA, fB, W1r_ab, b1_ab, W1r_ba, b1_ba):
    def body(fA_r, fB_r, wra_r, ba_r, wrb_r, bb_r, rB1_r, rA1_r):
        rB1_r[...] = jnp.dot(fB_r[...], wra_r[...],
                             preferred_element_type=jnp.float32) + ba_r[...]
        rA1_r[...] = jnp.dot(fA_r[...], wrb_r[...],
                             preferred_element_type=jnp.float32) + bb_r[...]

    out = [jax.ShapeDtypeStruct((N, H), jnp.float32)] * 2
    return pl.pallas_call(body, out_shape=out)(
        fA, fB, W1r_ab, b1_ab.reshape(1, H), W1r_ba, b1_ba.reshape(1, H))


def _stage_mid_one(pX, icX, rX1, g1, be1, Wl_next):
    def body(pX_r, icX_r, rX1_r, g1_r, be1_r, wl_r, tX2_r, hX_r):
        hX = pX_r[:N, :] * icX_r[...] + rX1_r[...]
        hX = _bn_lrelu(hX, g1_r[...], be1_r[...])
        hX_r[...] = hX
        tX2_r[...] = jnp.dot(hX, wl_r[...], preferred_element_type=jnp.float32)

    out = [jax.ShapeDtypeStruct((N, H), jnp.float32)] * 2
    return pl.pallas_call(body, out_shape=out)(
        pX, icX, rX1, g1.reshape(1, H), be1.reshape(1, H), Wl_next)


def _rterms2(hB, hA, W2r_ab, b2_ab, W2r_ba, b2_ba):
    def body(hB_r, hA_r, wra_r, ba_r, wrb_r, bb_r, rB2_r, rA2_r):
        rB2_r[...] = jnp.dot(hB_r[...], wra_r[...],
                             preferred_element_type=jnp.float32) + ba_r[...]
        rA2_r[...] = jnp.dot(hA_r[...], wrb_r[...],
                             preferred_element_type=jnp.float32) + bb_r[...]

    out = [jax.ShapeDtypeStruct((N, H), jnp.float32)] * 2
    return pl.pallas_call(body, out_shape=out)(
        hB, hA, W2r_ab, b2_ab.reshape(1, H), W2r_ba, b2_ba.reshape(1, H))


def _post_head(pB2, pA2, icB, icA, rB2, rA2, g2_B, be2_B, g2_A, be2_A,
               batch_B, batch_A, Wg1, bg1, Wg2, bg2, Wg3, bg3):
    def body(pB_r, pA_r, icB_r, icA_r, rB2_r, rA2_r, g2B_r, be2B_r, g2A_r,
             be2A_r, bB_r, bA_r, wg1_r, b1_r, wg2_r, b2_r, wg3_r, b3_r,
             out_r):
        h2B = pB_r[:N, :] * icB_r[...] + rB2_r[...]
        h2B = _bn_lrelu(h2B, g2B_r[...], be2B_r[...])
        h2A = pA_r[:N, :] * icA_r[...] + rA2_r[...]
        h2A = _bn_lrelu(h2A, g2A_r[...], be2A_r[...])
        gids = lax.broadcasted_iota(jnp.int32, (G, N), 0)
        ohB = (bB_r[...] == gids).astype(jnp.float32)
        ohA = (bA_r[...] == gids).astype(jnp.float32)
        gB = jnp.dot(ohB, h2B, preferred_element_type=jnp.float32) / jnp.maximum(
            jnp.sum(ohB, axis=1, keepdims=True), 1.0)
        gA = jnp.dot(ohA, h2A, preferred_element_type=jnp.float32) / jnp.maximum(
            jnp.sum(ohA, axis=1, keepdims=True), 1.0)
        emb = jnp.concatenate([gA, gB], axis=1)
        p = jnp.dot(emb, wg1_r[...], preferred_element_type=jnp.float32) + b1_r[...]
        p = jnp.dot(p, wg2_r[...], preferred_element_type=jnp.float32) + b2_r[...]
        p = jnp.dot(p, wg3_r[...], preferred_element_type=jnp.float32) + b3_r[...]
        pm = jnp.max(p, axis=1, keepdims=True)
        lse = jnp.log(jnp.sum(jnp.exp(p - pm), axis=1, keepdims=True)) + pm
        out_r[...] = p - lse

    out = jax.ShapeDtypeStruct((G, 8), jnp.float32)
    return pl.pallas_call(body, out_shape=out)(
        pB2, pA2, icB, icA, rB2, rA2, g2_B.reshape(1, H), be2_B.reshape(1, H),
        g2_A.reshape(1, H), be2_A.reshape(1, H),
        batch_B.reshape(1, N).astype(jnp.int32),
        batch_A.reshape(1, N).astype(jnp.int32),
        Wg1, bg1.reshape(1, H), Wg2, bg2.reshape(1, 16), Wg3, bg3.reshape(1, 8))


def kernel(node_feature_A, node_feature_B, edge_index_ab, edge_index_ba,
           batch_A, batch_B, W1l_ab, W1r_ab, b1_ab, W2l_ab, W2r_ab, b2_ab,
           W1l_ba, W1r_ba, b1_ba, W2l_ba, W2r_ba, b2_ba, g1_A, be1_A, g2_A,
           be2_A, g1_B, be1_B, g2_B, be2_B, Wg1, bg1, Wg2, bg2, Wg3, bg3):
    fA, fB = node_feature_A, node_feature_B
    sAB, dAB = _pad_edges(edge_index_ab)
    sBA, dBA = _pad_edges(edge_index_ba)
    zero_nh = jnp.zeros((NPAD, H), jnp.float32)
    zero_n = jnp.zeros((NPAD,), jnp.float32)

    tA1, tB1 = _tables1(fA, fB, W1l_ab, W1l_ba)
    pB, pA, cB, cA = _seg_with_counts(tA1, tB1, sAB, dAB, sBA, dBA,
                                      zero_nh, zero_n)
    rB1, rA1 = _rterms1(fA, fB, W1r_ab, b1_ab, W1r_ba, b1_ba)
    icB = jnp.broadcast_to(
        (1.0 / jnp.maximum(cB, 1.0))[:N, None], (N, H))
    icA = jnp.broadcast_to(
        (1.0 / jnp.maximum(cA, 1.0))[:N, None], (N, H))
    tB2, hB = _stage_mid_one(pB, icB, rB1, g1_B, be1_B, W2l_ba)
    tA2, hA = _stage_mid_one(pA, icA, rA1, g1_A, be1_A, W2l_ab)

    pB2, pA2 = _seg_no_counts(tA2, tB2, sAB, dAB, sBA, dBA, zero_nh, zero_n)
    rB2, rA2 = _rterms2(hB, hA, W2r_ab, b2_ab, W2r_ba, b2_ba)
    return _post_head(pB2, pA2, icB, icA, rB2, rA2, g2_B, be2_B, g2_A,
                      be2_A, batch_B, batch_A, Wg1, bg1, Wg2, bg2, Wg3, bg3)


# speedup vs baseline: 10.6520x; 1.0011x over previous
import functools

import jax
import jax.numpy as jnp
from jax import lax
from jax.experimental import pallas as pl
from jax.experimental.pallas import tpu as pltpu
from jax.experimental.pallas import tpu_sc as plsc

N = 10000
D = 128
H = 64
E = 320000
G = 64
EPS = 1e-5

NC = 2
NS = 16
NW = NC * NS
CH = 128
Q = 1
NB = 4 * Q
TILE_E = 20480
EPAD = NS * TILE_E
ROWS_PER_TILE = TILE_E // CH
OCTETS = ROWS_PER_TILE // Q // 8
NSTAGE = N // NS
NPAD = 10112
RPT = NPAD // NS


def _make_seg_kernel(with_counts):
    mesh = plsc.VectorSubcoreMesh(
        core_axis_name="c", subcore_axis_name="s", num_cores=NC, num_subcores=NS
    )
    outs = [
        jax.ShapeDtypeStruct((NPAD, H), jnp.float32),
        jax.ShapeDtypeStruct((NPAD, H), jnp.float32),
    ]
    scratch = [
        pltpu.VMEM((2, 4 * Q, CH), jnp.int32),
        pltpu.VMEM((2, 4 * Q, CH), jnp.int32),
        pltpu.VMEM((NB, CH, H), jnp.float32),
        pltpu.VMEM_SHARED((NPAD, H), jnp.float32),
        pltpu.VMEM_SHARED((N, H), jnp.float32),
        pltpu.SemaphoreType.DMA,
        pltpu.SemaphoreType.DMA,
        pltpu.SemaphoreType.DMA,
        pltpu.SemaphoreType.DMA,
        pltpu.SemaphoreType.DMA,
        pltpu.SemaphoreType.DMA,
        pltpu.SemaphoreType.DMA,
        pltpu.SemaphoreType.DMA,
    ]
    if with_counts:
        outs += [
            jax.ShapeDtypeStruct((NPAD,), jnp.float32),
            jax.ShapeDtypeStruct((NPAD,), jnp.float32),
        ]
        scratch += [
            pltpu.VMEM((CH,), jnp.float32),
            pltpu.VMEM_SHARED((NPAD,), jnp.float32),
            pltpu.SemaphoreType.DMA,
        ]

    def body(*refs):
        (tA, tB, sAB, dAB, sBA, dBA, zero_nh, zero_n) = refs[:8]
        (pB, pA) = refs[8:10]
        if with_counts:
            (cB, cA) = refs[10:12]
            (isv, idv, rows, acc, tbl, gs0, gs1, gs2, gs3, ss0, ss1, ss2,
             ss3, ones_v, cnt, csem) = refs[12:]
        else:
            (isv, idv, rows, acc, tbl, gs0, gs1, gs2, gs3,
             ss0, ss1, ss2, ss3) = refs[10:]
            cnt = None

        cid = lax.axis_index("c")
        sid = lax.axis_index("s")

        @pl.when(sid == 0)
        def _zero():
            pltpu.sync_copy(zero_nh, acc)
            if with_counts:
                pltpu.sync_copy(zero_n, cnt)

        if with_counts:
            for i in range(CH // 16):
                ones_v[pl.ds(i * 16, 16)] = jnp.ones((16,), jnp.float32)

        tsl = pl.ds(sid * NSTAGE, NSTAGE)

        @pl.when(cid == 0)
        def _stage_a():
            pltpu.sync_copy(tA.at[tsl], tbl.at[tsl])

        @pl.when(cid == 1)
        def _stage_b():
            pltpu.sync_copy(tB.at[tsl], tbl.at[tsl])

        plsc.subcore_barrier()

        def run_dir(src2d, dst2d):
            base = sid * ROWS_PER_TILE
            gs = [gs0, gs1, gs2, gs3]
            ss = [ss0, ss1, ss2, ss3]

            def ldbatch(b, p):
                r0 = base + b * 4 * Q
                pltpu.sync_copy(src2d.at[pl.ds(r0, 4 * Q)], isv.at[p])
                pltpu.sync_copy(dst2d.at[pl.ds(r0, 4 * Q)], idv.at[p])

            def gissue(p, row, qj):
                for q in range(Q):
                    pltpu.async_copy(tbl.at[isv.at[p, row + q]],
                                     rows.at[qj * Q + q], gs[qj])

            def gwait(qj):
                for q in range(Q):
                    pltpu.make_async_copy(tbl.at[isv.at[0, 0]],
                                          rows.at[qj * Q + q], gs[qj]).wait()

            def sissue(p, row, qj):
                for q in range(Q):
                    pltpu.async_copy(rows.at[qj * Q + q],
                                     acc.at[idv.at[p, row + q]], ss[qj],
                                     add=True)
                    if with_counts:
                        pltpu.async_copy(ones_v, cnt.at[idv.at[p, row + q]],
                                         csem, add=True)

            def swait(qj):
                for q in range(Q):
                    pltpu.make_async_copy(rows.at[qj * Q + q],
                                          acc.at[idv.at[0, 0]], ss[qj]).wait()

            def cwait(n):
                for _ in range(n):
                    pltpu.make_async_copy(ones_v, cnt.at[idv.at[0, 0]],
                                          csem).wait()

            def step(o, u, do_swait, do_gnext, do_ld):
                pb = (u // 4) % 2
                row = (u % 4) * Q
                qj = u % 4
                gwait(qj)
                sissue(pb, row, qj)
                if do_swait:
                    swait((u + 2) % 4)
                    if with_counts:
                        cwait(Q)
                if do_gnext:
                    gissue(((u + 2) // 4) % 2, ((u + 2) % 4) * Q, (u + 2) % 4)
                if do_ld and u % 4 == 1:
                    ldbatch(2 * o + (u + 3) // 4, ((u + 3) // 4) % 2)

            ldbatch(0, 0)
            ldbatch(1, 1)
            gissue(0, 0, 0)
            gissue(0, Q, 1)

            zero_o = jnp.int32(0)
            for u in range(8):
                step(zero_o, u, u >= 2, True, u == 5)

            def outer(o, carry):
                for u in range(8):
                    step(o, u, True, True, True)
                return carry
            lax.fori_loop(1, OCTETS - 1, outer, 0)

            last_o = jnp.int32(OCTETS - 1)
            for u in range(8):
                step(last_o, u, True, u < 6, u == 1)

            swait(2)
            swait(3)
            if with_counts:
                cwait(2 * Q)

        @pl.when(cid == 0)
        def _dir_ab():
            run_dir(sAB, dAB)

        @pl.when(cid == 1)
        def _dir_ba():
            run_dir(sBA, dBA)

        plsc.subcore_barrier()

        sl = pl.ds(sid * RPT, RPT)

        @pl.when(cid == 0)
        def _wb_b():
            pltpu.sync_copy(acc.at[sl], pB.at[sl])
            if with_counts:
                @pl.when(sid == 0)
                def _wc_b():
                    pltpu.sync_copy(cnt, cB)

        @pl.when(cid == 1)
        def _wb_a():
            pltpu.sync_copy(acc.at[sl], pA.at[sl])
            if with_counts:
                @pl.when(sid == 0)
                def _wc_a():
                    pltpu.sync_copy(cnt, cA)

    return pl.kernel(body, out_type=outs, mesh=mesh, scratch_types=scratch,
                     compiler_params=pltpu.CompilerParams(
                         use_tc_tiling_on_sc=False),
                     name="seg_sum_cnt" if with_counts else "seg_sum")


_seg_with_counts = _make_seg_kernel(True)
_seg_no_counts = _make_seg_kernel(False)


def _pad_edges(ei):
    src = jnp.concatenate(
        [ei[0].astype(jnp.int32), jnp.zeros((EPAD - E,), jnp.int32)])
    dst = jnp.concatenate(
        [ei[1].astype(jnp.int32), jnp.full((EPAD - E,), N, jnp.int32)])
    return src.reshape(EPAD // CH, CH), dst.reshape(EPAD // CH, CH)


def _plrelu(x):
    return jnp.where(x >= 0, x, 0.01 * x)


def _bn_lrelu(x, g, b):
    m = jnp.mean(x, axis=0, keepdims=True)
    v = jnp.mean((x - m) * (x - m), axis=0, keepdims=True)
    return _plrelu((x - m) * lax.rsqrt(v + EPS) * g + b)


_TC_PARAMS = pltpu.CompilerParams(vmem_limit_bytes=110 * 2**20)

def _tables1(fA, fB, W1l_ab, W1l_ba):
    def body(fA_r, fB_r, wla_r, wlb_r, tA1_r, tB1_r):
        tA1_r[...] = jnp.dot(fA_r[...], wla_r[...],
                             preferred_element_type=jnp.float32)
        tB1_r[...] = jnp.dot(fB_r[...], wlb_r[...],
                             preferred_element_type=jnp.float32)

    out = [jax.ShapeDtypeStruct((N, H), jnp.float32)] * 2
    return pl.pallas_call(body, out_shape=out)(fA, fB, W1l_ab, W1l_ba)


def _rterms1(fA, fB, W1r_ab, b1_ab, W1r_ba, b1_ba):
    def body(fA_r, fB_r, wra_r, ba_r, wrb_r, bb_r, rB1_r, rA1_r):
        rB1_r[...] = jnp.dot(fB_r[...], wra_r[...],
                             preferred_element_type=jnp.float32) + ba_r[...]
        rA1_r[...] = jnp.dot(fA_r[...], wrb_r[...],
                             preferred_element_type=jnp.float32) + bb_r[...]

    out = [jax.ShapeDtypeStruct((N, H), jnp.float32)] * 2
    return pl.pallas_call(body, out_shape=out)(
        fA, fB, W1r_ab, b1_ab.reshape(1, H), W1r_ba, b1_ba.reshape(1, H))


def _stage_mid_one(pX, icX, rX1, g1, be1, Wl_next):
    def body(pX_r, icX_r, rX1_r, g1_r, be1_r, wl_r, tX2_r, hX_r):
        hX = pX_r[:N, :] * icX_r[...] + rX1_r[...]
        hX = _bn_lrelu(hX, g1_r[...], be1_r[...])
        hX_r[...] = hX
        tX2_r[...] = jnp.dot(hX, wl_r[...], preferred_element_type=jnp.float32)

    out = [jax.ShapeDtypeStruct((N, H), jnp.float32)] * 2
    return pl.pallas_call(body, out_shape=out)(
        pX, icX, rX1, g1.reshape(1, H), be1.reshape(1, H), Wl_next)


def _rterms2(hB, hA, W2r_ab, b2_ab, W2r_ba, b2_ba):
    def body(hB_r, hA_r, wra_r, ba_r, wrb_r, bb_r, rB2_r, rA2_r):
        rB2_r[...] = jnp.dot(hB_r[...], wra_r[...],
                             preferred_element_type=jnp.float32) + ba_r[...]
        rA2_r[...] = jnp.dot(hA_r[...], wrb_r[...],
                             preferred_element_type=jnp.float32) + bb_r[...]

    out = [jax.ShapeDtypeStruct((N, H), jnp.float32)] * 2
    return pl.pallas_call(body, out_shape=out)(
        hB, hA, W2r_ab, b2_ab.reshape(1, H), W2r_ba, b2_ba.reshape(1, H))


def _post_head(pB2, pA2, icB, icA, rB2, rA2, g2_B, be2_B, g2_A, be2_A,
               batch_B, batch_A, Wg1, bg1, Wg2, bg2, Wg3, bg3):
    def body(pB_r, pA_r, icB_r, icA_r, rB2_r, rA2_r, g2B_r, be2B_r, g2A_r,
             be2A_r, bB_r, bA_r, wg1_r, b1_r, wg2_r, b2_r, wg3_r, b3_r,
             out_r):
        h2B = pB_r[:N, :] * icB_r[...] + rB2_r[...]
        h2B = _bn_lrelu(h2B, g2B_r[...], be2B_r[...])
        h2A = pA_r[:N, :] * icA_r[...] + rA2_r[...]
        h2A = _bn_lrelu(h2A, g2A_r[...], be2A_r[...])
        gids = lax.broadcasted_iota(jnp.int32, (G, N), 0)
        ohB = (bB_r[...] == gids).astype(jnp.float32)
        ohA = (bA_r[...] == gids).astype(jnp.float32)
        gB = jnp.dot(ohB, h2B, preferred_element_type=jnp.float32) / jnp.maximum(
            jnp.sum(ohB, axis=1, keepdims=True), 1.0)
        gA = jnp.dot(ohA, h2A, preferred_element_type=jnp.float32) / jnp.maximum(
            jnp.sum(ohA, axis=1, keepdims=True), 1.0)
        emb = jnp.concatenate([gA, gB], axis=1)
        p = jnp.dot(emb, wg1_r[...], preferred_element_type=jnp.float32) + b1_r[...]
        p = jnp.dot(p, wg2_r[...], preferred_element_type=jnp.float32) + b2_r[...]
        p = jnp.dot(p, wg3_r[...], preferred_element_type=jnp.float32) + b3_r[...]
        pm = jnp.max(p, axis=1, keepdims=True)
        lse = jnp.log(jnp.sum(jnp.exp(p - pm), axis=1, keepdims=True)) + pm
        out_r[...] = p - lse

    out = jax.ShapeDtypeStruct((G, 8), jnp.float32)
    return pl.pallas_call(body, out_shape=out)(
        pB2, pA2, icB, icA, rB2, rA2, g2_B.reshape(1, H), be2_B.reshape(1, H),
        g2_A.reshape(1, H), be2_A.reshape(1, H),
        batch_B.reshape(1, N).astype(jnp.int32),
        batch_A.reshape(1, N).astype(jnp.int32),
        Wg1, bg1.reshape(1, H), Wg2, bg2.reshape(1, 16), Wg3, bg3.reshape(1, 8))


def kernel(node_feature_A, node_feature_B, edge_index_ab, edge_index_ba,
           batch_A, batch_B, W1l_ab, W1r_ab, b1_ab, W2l_ab, W2r_ab, b2_ab,
           W1l_ba, W1r_ba, b1_ba, W2l_ba, W2r_ba, b2_ba, g1_A, be1_A, g2_A,
           be2_A, g1_B, be1_B, g2_B, be2_B, Wg1, bg1, Wg2, bg2, Wg3, bg3):
    fA, fB = node_feature_A, node_feature_B
    sAB, dAB = _pad_edges(edge_index_ab)
    sBA, dBA = _pad_edges(edge_index_ba)
    zero_nh = jnp.zeros((NPAD, H), jnp.float32)
    zero_n = jnp.zeros((NPAD,), jnp.float32)

    tA1, tB1 = _tables1(fA, fB, W1l_ab, W1l_ba)
    pB, pA, cB, cA = _seg_with_counts(tA1, tB1, sAB, dAB, sBA, dBA,
                                      zero_nh, zero_n)
    rB1, rA1 = _rterms1(fA, fB, W1r_ab, b1_ab, W1r_ba, b1_ba)
    icB = jnp.broadcast_to(
        (1.0 / jnp.maximum(cB, 1.0))[:N, None], (N, H))
    icA = jnp.broadcast_to(
        (1.0 / jnp.maximum(cA, 1.0))[:N, None], (N, H))
    tB2, hB = _stage_mid_one(pB, icB, rB1, g1_B, be1_B, W2l_ba)
    tA2, hA = _stage_mid_one(pA, icA, rA1, g1_A, be1_A, W2l_ab)

    pB2, pA2 = _seg_no_counts(tA2, tB2, sAB, dAB, sBA, dBA, zero_nh, zero_n)
    rB2, rA2 = _rterms2(hB, hA, W2r_ab, b2_ab, W2r_ba, b2_ba)
    return _post_head(pB2, pA2, icB, icA, rB2, rA2, g2_B, be2_B, g2_A,
                      be2_A, batch_B, batch_A, Wg1, bg1, Wg2, bg2, Wg3, bg3)
